# baseline (speedup 1.0000x reference)
import functools

import jax
import jax.numpy as jnp
from jax import lax
from jax.experimental import pallas as pl
from jax.experimental.pallas import tpu as pltpu
from jax.experimental.pallas import tpu_sc as plsc

NC = 2
NS = 16
L = 16
NW = NC * NS

EMB_SZ = 64
BATCH = 16384
B_PER_W = BATCH // NW
CHUNK = 128
N_CHUNKS = B_PER_W // CHUNK
N_GROUPS = B_PER_W // L


def _mf_kernel(remb_hbm, pemb_hbm, rbias_hbm, pbias_hbm, pid_hbm, rid_hbm,
               out_hbm,
               rid_v, pid_v, re_v, pe_v, rb_v, pb_v, out_v,
               sem_re, sem_pe, sem_rb, sem_pb):
    wid = lax.axis_index("s") * NC + lax.axis_index("c")
    base = wid * B_PER_W

    for j in range(N_CHUNKS):
        pltpu.sync_copy(rid_hbm.at[pl.ds(base + j * CHUNK, CHUNK)], rid_v.at[j])
        pltpu.sync_copy(pid_hbm.at[pl.ds(base + j * CHUNK, CHUNK)], pid_v.at[j])

    copies = []
    for j in range(N_CHUNKS):
        sl = pl.ds(j * CHUNK, CHUNK)
        copies.append(pltpu.async_copy(remb_hbm.at[rid_v.at[j]], re_v.at[sl], sem_re))
        copies.append(pltpu.async_copy(pemb_hbm.at[pid_v.at[j]], pe_v.at[sl], sem_pe))
        copies.append(pltpu.async_copy(rbias_hbm.at[rid_v.at[j]], rb_v.at[sl], sem_rb))
        copies.append(pltpu.async_copy(pbias_hbm.at[pid_v.at[j]], pb_v.at[sl], sem_pb))
    for c in copies:
        c.wait()

    def group_body(g, carry):
        rows = lax.iota(jnp.int32, L) + g * L
        acc = rb_v[pl.ds(g * L, L)] + pb_v[pl.ds(g * L, L)]
        for d in range(EMB_SZ):
            dv = jnp.full((L,), d, jnp.int32)
            a = plsc.load_gather(re_v, [rows, dv])
            b = plsc.load_gather(pe_v, [rows, dv])
            acc = acc + a * b
        out_v[pl.ds(g * L, L)] = acc
        return carry

    lax.fori_loop(0, N_GROUPS, group_body, 0)

    pltpu.sync_copy(out_v, out_hbm.at[pl.ds(base, B_PER_W)])


@jax.jit
def _mf(reviewer_emb, product_emb, reviewer_bias, product_bias, product_id,
        reviewer_id):
    mesh = plsc.VectorSubcoreMesh(core_axis_name="c", subcore_axis_name="s")
    return pl.kernel(
        _mf_kernel,
        out_type=jax.ShapeDtypeStruct((BATCH,), jnp.float32),
        mesh=mesh,
        compiler_params=pltpu.CompilerParams(
            needs_layout_passes=False, use_tc_tiling_on_sc=False),
        scratch_types=[
            pltpu.VMEM((N_CHUNKS, CHUNK), jnp.int32),
            pltpu.VMEM((N_CHUNKS, CHUNK), jnp.int32),
            pltpu.VMEM((B_PER_W, EMB_SZ), jnp.float32),
            pltpu.VMEM((B_PER_W, EMB_SZ), jnp.float32),
            pltpu.VMEM((B_PER_W,), jnp.float32),
            pltpu.VMEM((B_PER_W,), jnp.float32),
            pltpu.VMEM((B_PER_W,), jnp.float32),
            pltpu.SemaphoreType.DMA,
            pltpu.SemaphoreType.DMA,
            pltpu.SemaphoreType.DMA,
            pltpu.SemaphoreType.DMA,
        ],
    )(reviewer_emb, product_emb, reviewer_bias, product_bias, product_id,
      reviewer_id)


def kernel(reviewer_emb, product_emb, reviewer_bias, product_bias, product_id,
           reviewer_id):
    return _mf(reviewer_emb, product_emb,
               reviewer_bias.reshape(-1), product_bias.reshape(-1),
               product_id.astype(jnp.int32), reviewer_id.astype(jnp.int32))
